# speedup vs baseline: 2.8242x; 2.8242x over previous
import jax
import jax.numpy as jnp
import numpy as np
from jax.experimental import pallas as pl
from jax.experimental.pallas import tpu as pltpu

_B, _C, _D, _H, _W = 2, 3, 128, 128, 128
_SLICES = _B * _C
_SLICE_ELEMS = _D * _H * _W
_N_TOPK = max(1, round(_SLICE_ELEMS * 10 / 100))

_ROWS, _COLS = 768, 16384
_BLK_ROWS = 64


def _loss_body(x_ref, t_ref, o_ref):
    x = x_ref[...]
    t = t_ref[...]
    o_ref[...] = jnp.maximum(x, 0.0) - x * t + jnp.log1p(jnp.exp(-jnp.abs(x)))


def _loss_buffer(net_output, target):
    x = net_output.reshape(_ROWS, _COLS)
    t = target.reshape(_ROWS, _COLS)
    return pl.pallas_call(
        _loss_body,
        grid=(_ROWS // _BLK_ROWS,),
        in_specs=[
            pl.BlockSpec((_BLK_ROWS, _COLS), lambda i: (i, 0)),
            pl.BlockSpec((_BLK_ROWS, _COLS), lambda i: (i, 0)),
        ],
        out_specs=pl.BlockSpec((_BLK_ROWS, _COLS), lambda i: (i, 0)),
        out_shape=jax.ShapeDtypeStruct((_ROWS, _COLS), jnp.float32),
    )(x, t)


def kernel(net_output, target):
    loss = _loss_buffer(net_output, target)
    loss = loss.reshape(_SLICES, _SLICE_ELEMS)
    vals, _ = jax.lax.top_k(loss, _N_TOPK)
    return vals.mean(-1).mean()
